# baseline (speedup 1.0000x reference)
import jax
import jax.numpy as jnp
from jax.experimental import pallas as pl
from jax.experimental.pallas import tpu as pltpu

E = 16
TOPK = 2


def _moe_body(x_ref, gate_ref, wg_ref, wu_ref, wd_ref, swg_ref, swu_ref, swd_ref,
              out_ref, comb_ref):
    e = pl.program_id(0)
    x = x_ref[...]

    @pl.when(e == 0)
    def _init():
        logits = jnp.dot(x, gate_ref[...], preferred_element_type=jnp.float32)
        ecols = jax.lax.broadcasted_iota(jnp.int32, logits.shape, 1)
        l1 = jnp.max(logits, axis=-1, keepdims=True)
        i1 = jnp.min(jnp.where(logits == l1, ecols, E), axis=-1, keepdims=True)
        masked = jnp.where(ecols == i1, -jnp.inf, logits)
        l2 = jnp.max(masked, axis=-1, keepdims=True)
        i2 = jnp.min(jnp.where(masked == l2, ecols, E), axis=-1, keepdims=True)
        w1 = 1.0 / (1.0 + jnp.exp(l2 - l1))
        w2 = 1.0 - w1
        comb_ref[...] = jnp.where(ecols == i1, w1, 0.0) + jnp.where(ecols == i2, w2, 0.0)

        xb0 = x.astype(jnp.bfloat16)
        sg = jnp.dot(xb0, swg_ref[...], preferred_element_type=jnp.float32)
        su = jnp.dot(xb0, swu_ref[...], preferred_element_type=jnp.float32)
        sh = ((sg * jax.nn.sigmoid(sg)) * su).astype(jnp.bfloat16)
        out_ref[...] = jnp.dot(sh, swd_ref[...], preferred_element_type=jnp.float32)

    xb = x.astype(jnp.bfloat16)
    g = jnp.dot(xb, wg_ref[0], preferred_element_type=jnp.float32)
    u = jnp.dot(xb, wu_ref[0], preferred_element_type=jnp.float32)
    h = ((g * jax.nn.sigmoid(g)) * u).astype(jnp.bfloat16)
    eo = jnp.dot(h, wd_ref[0], preferred_element_type=jnp.float32)
    ecols = jax.lax.broadcasted_iota(jnp.int32, comb_ref.shape, 1)
    coef = jnp.sum(jnp.where(ecols == e, comb_ref[...], 0.0), axis=-1, keepdims=True)
    out_ref[...] += coef * eo


def kernel(hidden_states, gate_w, expert_wg, expert_wu, expert_wd,
           shared_wg, shared_wu, shared_wd):
    orig_shape = hidden_states.shape
    D = orig_shape[-1]
    x = hidden_states.reshape(-1, D)
    T = x.shape[0]
    expert_wg = expert_wg.astype(jnp.bfloat16)
    expert_wu = expert_wu.astype(jnp.bfloat16)
    expert_wd = expert_wd.astype(jnp.bfloat16)
    shared_wg = shared_wg.astype(jnp.bfloat16)
    shared_wu = shared_wu.astype(jnp.bfloat16)
    shared_wd = shared_wd.astype(jnp.bfloat16)
    F = expert_wg.shape[-1]
    SF = shared_wg.shape[-1]

    out = pl.pallas_call(
        _moe_body,
        grid=(E,),
        in_specs=[
            pl.BlockSpec((T, D), lambda e: (0, 0)),
            pl.BlockSpec((D, E), lambda e: (0, 0)),
            pl.BlockSpec((1, D, F), lambda e: (e, 0, 0)),
            pl.BlockSpec((1, D, F), lambda e: (e, 0, 0)),
            pl.BlockSpec((1, F, D), lambda e: (e, 0, 0)),
            pl.BlockSpec((D, SF), lambda e: (0, 0)),
            pl.BlockSpec((D, SF), lambda e: (0, 0)),
            pl.BlockSpec((SF, D), lambda e: (0, 0)),
        ],
        out_specs=pl.BlockSpec((T, D), lambda e: (0, 0)),
        out_shape=jax.ShapeDtypeStruct((T, D), jnp.float32),
        scratch_shapes=[pltpu.VMEM((T, E), jnp.float32)],
        compiler_params=pltpu.CompilerParams(
            dimension_semantics=("arbitrary",),
        ),
    )(x, gate_w, expert_wg, expert_wu, expert_wd, shared_wg, shared_wu, shared_wd)

    return out.reshape(orig_shape)


# speedup vs baseline: 1.4213x; 1.4213x over previous
import jax
import jax.numpy as jnp
from jax.experimental import pallas as pl
from jax.experimental.pallas import tpu as pltpu

E = 16
TOPK = 2


def _moe_body(x_ref, gate_ref, wg_ref, wu_ref, wd_ref, swg_ref, swu_ref, swd_ref,
              out_ref, comb_ref):
    e = pl.program_id(0)
    x = x_ref[...]

    @pl.when(e == 0)
    def _init():
        logits = jnp.dot(x, gate_ref[...], preferred_element_type=jnp.float32)
        ecols = jax.lax.broadcasted_iota(jnp.int32, logits.shape, 1)
        l1 = jnp.max(logits, axis=-1, keepdims=True)
        i1 = jnp.min(jnp.where(logits == l1, ecols, E), axis=-1, keepdims=True)
        masked = jnp.where(ecols == i1, -jnp.inf, logits)
        l2 = jnp.max(masked, axis=-1, keepdims=True)
        i2 = jnp.min(jnp.where(masked == l2, ecols, E), axis=-1, keepdims=True)
        w1 = 1.0 / (1.0 + jnp.exp(l2 - l1))
        w2 = 1.0 - w1
        comb_ref[...] = jnp.where(ecols == i1, w1, 0.0) + jnp.where(ecols == i2, w2, 0.0)

        xb0 = x.astype(jnp.bfloat16)
        sg = jnp.dot(xb0, swg_ref[...].astype(jnp.bfloat16),
                     preferred_element_type=jnp.float32)
        su = jnp.dot(xb0, swu_ref[...].astype(jnp.bfloat16),
                     preferred_element_type=jnp.float32)
        sh = ((sg * jax.nn.sigmoid(sg)) * su).astype(jnp.bfloat16)
        out_ref[...] = jnp.dot(sh, swd_ref[...].astype(jnp.bfloat16),
                               preferred_element_type=jnp.float32)

    xb = x.astype(jnp.bfloat16)
    g = jnp.dot(xb, wg_ref[0].astype(jnp.bfloat16), preferred_element_type=jnp.float32)
    u = jnp.dot(xb, wu_ref[0].astype(jnp.bfloat16), preferred_element_type=jnp.float32)
    h = ((g * jax.nn.sigmoid(g)) * u).astype(jnp.bfloat16)
    eo = jnp.dot(h, wd_ref[0].astype(jnp.bfloat16), preferred_element_type=jnp.float32)
    ecols = jax.lax.broadcasted_iota(jnp.int32, comb_ref.shape, 1)
    coef = jnp.sum(jnp.where(ecols == e, comb_ref[...], 0.0), axis=-1, keepdims=True)
    out_ref[...] += coef * eo


def kernel(hidden_states, gate_w, expert_wg, expert_wu, expert_wd,
           shared_wg, shared_wu, shared_wd):
    orig_shape = hidden_states.shape
    D = orig_shape[-1]
    x = hidden_states.reshape(-1, D)
    T = x.shape[0]
    F = expert_wg.shape[-1]
    SF = shared_wg.shape[-1]

    out = pl.pallas_call(
        _moe_body,
        grid=(E,),
        in_specs=[
            pl.BlockSpec((T, D), lambda e: (0, 0)),
            pl.BlockSpec((D, E), lambda e: (0, 0)),
            pl.BlockSpec((1, D, F), lambda e: (e, 0, 0)),
            pl.BlockSpec((1, D, F), lambda e: (e, 0, 0)),
            pl.BlockSpec((1, F, D), lambda e: (e, 0, 0)),
            pl.BlockSpec((D, SF), lambda e: (0, 0)),
            pl.BlockSpec((D, SF), lambda e: (0, 0)),
            pl.BlockSpec((SF, D), lambda e: (0, 0)),
        ],
        out_specs=pl.BlockSpec((T, D), lambda e: (0, 0)),
        out_shape=jax.ShapeDtypeStruct((T, D), jnp.float32),
        scratch_shapes=[pltpu.VMEM((T, E), jnp.float32)],
        compiler_params=pltpu.CompilerParams(
            dimension_semantics=("arbitrary",),
        ),
    )(x, gate_w, expert_wg, expert_wu, expert_wd, shared_wg, shared_wu, shared_wd)

    return out.reshape(orig_shape)


# speedup vs baseline: 1.6031x; 1.1279x over previous
import jax
import jax.numpy as jnp
from jax.experimental import pallas as pl
from jax.experimental.pallas import tpu as pltpu

E = 16
TOPK = 2


def _moe_body(x_ref, gate_ref, wg_ref, wu_ref, wd_ref, swg_ref, swu_ref, swd_ref,
              out_ref, comb_ref, xb_ref):
    e = pl.program_id(0)

    @pl.when(e == 0)
    def _init():
        x = x_ref[...]
        xb_ref[...] = x.astype(jnp.bfloat16)
        logits = jnp.dot(x, gate_ref[...], preferred_element_type=jnp.float32)
        ecols = jax.lax.broadcasted_iota(jnp.int32, logits.shape, 1)
        l1 = jnp.max(logits, axis=-1, keepdims=True)
        i1 = jnp.min(jnp.where(logits == l1, ecols, E), axis=-1, keepdims=True)
        masked = jnp.where(ecols == i1, -jnp.inf, logits)
        l2 = jnp.max(masked, axis=-1, keepdims=True)
        i2 = jnp.min(jnp.where(masked == l2, ecols, E), axis=-1, keepdims=True)
        w1 = 1.0 / (1.0 + jnp.exp(l2 - l1))
        w2 = 1.0 - w1
        comb_ref[...] = jnp.where(ecols == i1, w1, 0.0) + jnp.where(ecols == i2, w2, 0.0)

        xb0 = xb_ref[...]
        sg = jnp.dot(xb0, swg_ref[...].astype(jnp.bfloat16),
                     preferred_element_type=jnp.float32)
        su = jnp.dot(xb0, swu_ref[...].astype(jnp.bfloat16),
                     preferred_element_type=jnp.float32)
        sh = ((sg * jax.nn.sigmoid(sg)) * su).astype(jnp.bfloat16)
        out_ref[...] = jnp.dot(sh, swd_ref[...].astype(jnp.bfloat16),
                               preferred_element_type=jnp.float32)

    xb = xb_ref[...]
    g = jnp.dot(xb, wg_ref[0].astype(jnp.bfloat16), preferred_element_type=jnp.float32)
    u = jnp.dot(xb, wu_ref[0].astype(jnp.bfloat16), preferred_element_type=jnp.float32)
    ecols = jax.lax.broadcasted_iota(jnp.int32, comb_ref.shape, 1)
    coef = jnp.sum(jnp.where(ecols == e, comb_ref[...], 0.0), axis=-1, keepdims=True)
    h = (coef * (g * jax.nn.sigmoid(g)) * u).astype(jnp.bfloat16)
    out_ref[...] += jnp.dot(h, wd_ref[0].astype(jnp.bfloat16),
                            preferred_element_type=jnp.float32)


def kernel(hidden_states, gate_w, expert_wg, expert_wu, expert_wd,
           shared_wg, shared_wu, shared_wd):
    orig_shape = hidden_states.shape
    D = orig_shape[-1]
    x = hidden_states.reshape(-1, D)
    T = x.shape[0]
    F = expert_wg.shape[-1]
    SF = shared_wg.shape[-1]

    out = pl.pallas_call(
        _moe_body,
        grid=(E,),
        in_specs=[
            pl.BlockSpec((T, D), lambda e: (0, 0)),
            pl.BlockSpec((D, E), lambda e: (0, 0)),
            pl.BlockSpec((1, D, F), lambda e: (e, 0, 0)),
            pl.BlockSpec((1, D, F), lambda e: (e, 0, 0)),
            pl.BlockSpec((1, F, D), lambda e: (e, 0, 0)),
            pl.BlockSpec((D, SF), lambda e: (0, 0)),
            pl.BlockSpec((D, SF), lambda e: (0, 0)),
            pl.BlockSpec((SF, D), lambda e: (0, 0)),
        ],
        out_specs=pl.BlockSpec((T, D), lambda e: (0, 0)),
        out_shape=jax.ShapeDtypeStruct((T, D), jnp.float32),
        scratch_shapes=[pltpu.VMEM((T, E), jnp.float32),
                        pltpu.VMEM((T, D), jnp.bfloat16)],
        compiler_params=pltpu.CompilerParams(
            dimension_semantics=("arbitrary",),
        ),
    )(x, gate_w, expert_wg, expert_wu, expert_wd, shared_wg, shared_wu, shared_wd)

    return out.reshape(orig_shape)
